# Initial kernel scaffold; baseline (speedup 1.0000x reference)
import jax
import jax.numpy as jnp
from jax.experimental import pallas as pl


def kernel(query, support, W1, b1, W2, b2, ln_g, ln_b, W_ih, W_hh, b_ih, b_hh):
    raise NotImplementedError("write your pallas kernel here")


# speedup vs baseline: 5.5114x; 5.5114x over previous
import functools

import jax
import jax.numpy as jnp
from jax.experimental import pallas as pl
from jax.experimental.pallas import tpu as pltpu


def _body(q_ref, sup_ref, w1t_ref, b1_ref, w2t_ref, b2_ref, lng_ref, lnb_ref,
          wiht_ref, wh1t_ref, wh2t_ref, bias_ref, out_ref, *, D, H):
    sup = sup_ref[...]
    t = jnp.maximum(jnp.dot(sup, w1t_ref[...],
                            preferred_element_type=jnp.float32) + b1_ref[...], 0.0)
    t = jnp.dot(t, w2t_ref[...],
                preferred_element_type=jnp.float32) + b2_ref[...] + sup
    mu = jnp.mean(t, axis=1, keepdims=True)
    var = jnp.sum((t - mu) * (t - mu), axis=1, keepdims=True) / (D - 1)
    t = (t - mu) / (jnp.sqrt(var) + 1e-3) * lng_ref[...] + lnb_ref[...]
    sg = jnp.mean(t, axis=0, keepdims=True)
    sgc = jnp.dot(sg, wh2t_ref[...],
                  preferred_element_type=jnp.float32)

    q = q_ref[...]
    base = jnp.dot(q, wiht_ref[...],
                   preferred_element_type=jnp.float32) + bias_ref[...]

    def gates_to_hc(gates, c):
        i = jax.nn.sigmoid(gates[:, :H])
        f = jax.nn.sigmoid(gates[:, H:2 * H])
        g = jnp.tanh(gates[:, 2 * H:3 * H])
        o = jax.nn.sigmoid(gates[:, 3 * H:])
        c_new = f * c + i * g
        h_new = o * jnp.tanh(c_new)
        return h_new, c_new

    h_new, c = gates_to_hc(base, jnp.zeros((q.shape[0], H), jnp.float32))
    h = q + h_new[:, :D]
    for _ in range(3):
        gates = base + sgc + jnp.dot(h, wh1t_ref[...],
                                     preferred_element_type=jnp.float32)
        h_new, c = gates_to_hc(gates, c)
        h = q + h_new[:, :D]

    out_ref[...] = jnp.sum(h * sg, axis=1)


def kernel(query, support, W1, b1, W2, b2, ln_g, ln_b, W_ih, W_hh, b_ih, b_hh):
    B, D = query.shape
    H = W_hh.shape[1]
    G = W_ih.shape[0]
    Bt = 2048

    w_hhT = W_hh.T
    operands = (
        query,
        support,
        W1.T,
        b1.reshape(1, -1),
        W2.T,
        b2.reshape(1, -1),
        ln_g.reshape(1, -1),
        ln_b.reshape(1, -1),
        W_ih.T,
        w_hhT[:D],
        w_hhT[D:],
        (b_ih + b_hh).reshape(1, -1),
    )

    full = lambda shape: pl.BlockSpec(shape, lambda i: (0, 0))
    in_specs = [
        pl.BlockSpec((Bt, D), lambda i: (i, 0)),
        full(support.shape),
        full((D, 2 * D)),
        full((1, 2 * D)),
        full((2 * D, D)),
        full((1, D)),
        full((1, D)),
        full((1, D)),
        full((D, G)),
        full((D, G)),
        full((D, G)),
        full((1, G)),
    ]

    out = pl.pallas_call(
        functools.partial(_body, D=D, H=H),
        grid=(B // Bt,),
        in_specs=in_specs,
        out_specs=pl.BlockSpec((Bt,), lambda i: (i,)),
        out_shape=jax.ShapeDtypeStruct((B,), jnp.float32),
        compiler_params=pltpu.CompilerParams(
            dimension_semantics=("parallel",),
        ),
    )(*operands)
    return out


# speedup vs baseline: 7.1885x; 1.3043x over previous
import functools

import jax
import jax.numpy as jnp
from jax.experimental import pallas as pl
from jax.experimental.pallas import tpu as pltpu


def _tsig(x):
    return 0.5 * jnp.tanh(0.5 * x) + 0.5


def _body(q_ref, sup_ref, w1t_ref, b1_ref, w2t_ref, b2_ref, lng_ref, lnb_ref,
          wiht_ref, wh1t_ref, wh2t_ref, bias_ref, out_ref, *, D, H):
    sup = sup_ref[...]
    t = jnp.maximum(jnp.dot(sup, w1t_ref[...],
                            preferred_element_type=jnp.float32) + b1_ref[...], 0.0)
    t = jnp.dot(t, w2t_ref[...],
                preferred_element_type=jnp.float32) + b2_ref[...] + sup
    mu = jnp.mean(t, axis=1, keepdims=True)
    var = jnp.sum((t - mu) * (t - mu), axis=1, keepdims=True) / (D - 1)
    t = (t - mu) / (jnp.sqrt(var) + 1e-3) * lng_ref[...] + lnb_ref[...]
    sg = jnp.mean(t, axis=0, keepdims=True)
    sgc = jnp.dot(sg, wh2t_ref[...],
                  preferred_element_type=jnp.float32)

    q = q_ref[...]
    base = jnp.dot(q, wiht_ref[...],
                   preferred_element_type=jnp.float32) + bias_ref[...]
    base2 = base + sgc

    i = _tsig(base[:, :H])
    g = jnp.tanh(base[:, 2 * H:3 * H])
    o = _tsig(base[:, 3 * H:])
    c = i * g
    h = q + o * jnp.tanh(c[:, :D])

    for _ in range(2):
        gates = base2 + jnp.dot(h, wh1t_ref[...],
                                preferred_element_type=jnp.float32)
        i = _tsig(gates[:, :H])
        f = _tsig(gates[:, H:2 * H])
        g = jnp.tanh(gates[:, 2 * H:3 * H])
        o = _tsig(gates[:, 3 * H:])
        c = f * c + i * g
        h = q + o * jnp.tanh(c[:, :D])

    gates = base2 + jnp.dot(h, wh1t_ref[...],
                            preferred_element_type=jnp.float32)
    i = _tsig(gates[:, :D])
    f = _tsig(gates[:, H:H + D])
    g = jnp.tanh(gates[:, 2 * H:2 * H + D])
    o = _tsig(gates[:, 3 * H:])
    cD = f * c[:, :D] + i * g
    h = q + o * jnp.tanh(cD)

    out_ref[...] = jax.lax.dot_general(
        h, sg, (((1,), (1,)), ((), ())),
        preferred_element_type=jnp.float32)


def kernel(query, support, W1, b1, W2, b2, ln_g, ln_b, W_ih, W_hh, b_ih, b_hh):
    B, D = query.shape
    H = W_hh.shape[1]
    G = 3 * H + D
    Bt = 2048

    w_hhT = W_hh.T
    operands = (
        query,
        support,
        W1.T,
        b1.reshape(1, -1),
        W2.T,
        b2.reshape(1, -1),
        ln_g.reshape(1, -1),
        ln_b.reshape(1, -1),
        W_ih.T[:, :G],
        w_hhT[:D, :G],
        w_hhT[D:, :G],
        (b_ih + b_hh)[:G].reshape(1, -1),
    )

    full = lambda shape: pl.BlockSpec(shape, lambda i: (0, 0))
    in_specs = [
        pl.BlockSpec((Bt, D), lambda i: (i, 0)),
        full(support.shape),
        full((D, 2 * D)),
        full((1, 2 * D)),
        full((2 * D, D)),
        full((1, D)),
        full((1, D)),
        full((1, D)),
        full((D, G)),
        full((D, G)),
        full((D, G)),
        full((1, G)),
    ]

    out = pl.pallas_call(
        functools.partial(_body, D=D, H=H),
        grid=(B // Bt,),
        in_specs=in_specs,
        out_specs=pl.BlockSpec((Bt, 1), lambda i: (i, 0)),
        out_shape=jax.ShapeDtypeStruct((B, 1), jnp.float32),
        compiler_params=pltpu.CompilerParams(
            dimension_semantics=("parallel",),
        ),
    )(*operands)
    return out.reshape(B)
